# baseline (speedup 1.0000x reference)
import functools

import jax
import jax.numpy as jnp
from jax import lax
from jax.experimental import pallas as pl
from jax.experimental.pallas import tpu as pltpu
from jax.experimental.pallas import tpu_sc as plsc

EPS = 1e-12


@functools.cache
def _make_sc_gather(nb, seq, V, D):
    info = plsc.get_sparse_core_info()
    NC, NS = info.num_cores, info.num_subcores
    NW = NC * NS
    B = nb * seq
    assert B % NW == 0
    b_per_w = B // NW
    assert seq % b_per_w == 0
    CH = min(128, b_per_w)
    assert b_per_w % CH == 0
    n_ch = b_per_w // CH
    mesh = plsc.VectorSubcoreMesh(core_axis_name="c", subcore_axis_name="s")

    @functools.partial(
        pl.kernel,
        mesh=mesh,
        out_type=jax.ShapeDtypeStruct((B, D), jnp.float32),
        scratch_types=[
            pltpu.VMEM((b_per_w,), jnp.int32),
            pltpu.VMEM((b_per_w, D), jnp.float32),
        ]
        + [pltpu.SemaphoreType.DMA] * n_ch,
    )
    def gather(idx_hbm, table_hbm, out_hbm, idx_v, rows_v, *sems):
        wid = lax.axis_index("s") * NC + lax.axis_index("c")
        base = wid * b_per_w
        row = base // seq
        col = base % seq
        pltpu.sync_copy(idx_hbm.at[row, pl.ds(col, b_per_w)], idx_v)
        gathers = [
            pltpu.async_copy(
                table_hbm.at[idx_v.at[pl.ds(j * CH, CH)]],
                rows_v.at[pl.ds(j * CH, CH)],
                sems[j],
            )
            for j in range(n_ch)
        ]
        for g in gathers:
            g.wait()
        pltpu.sync_copy(rows_v, out_hbm.at[pl.ds(base, b_per_w)])

    return gather


@functools.cache
def _make_tc_proj(B, D, H, BT, NB):
    S = B // BT

    def body(x_hbm, g_ref, bt_ref, w_ref, b_ref, o_hbm, xb, ob, sx, so):
        def xcopy(s):
            return pltpu.make_async_copy(
                x_hbm.at[pl.ds(s * BT, BT)], xb.at[s % NB], sx.at[s % NB]
            )

        def ocopy(s):
            return pltpu.make_async_copy(
                ob.at[s % NB], o_hbm.at[pl.ds(s * BT, BT)], so.at[s % NB]
            )

        for s in range(min(NB, S)):
            xcopy(s).start()
        gam = g_ref[...][None, :]
        bet = bt_ref[...][None, :]
        w = w_ref[...]
        bias = b_ref[...][None, :]
        for s in range(S):
            xcopy(s).wait()
            x = xb[s % NB]
            mean = jnp.mean(x, axis=1, keepdims=True)
            xc = x - mean
            var = jnp.mean(xc * xc, axis=1, keepdims=True)
            xn = xc * lax.rsqrt(var + EPS) * gam + bet
            if s >= NB:
                ocopy(s - NB).wait()
            ob[s % NB] = (
                jnp.dot(xn, w, preferred_element_type=jnp.float32) + bias
            )
            ocopy(s).start()
            if s + NB < S:
                xcopy(s + NB).start()
        for s in range(max(S - NB, 0), S):
            ocopy(s).wait()

    return pl.pallas_call(
        body,
        in_specs=[
            pl.BlockSpec(memory_space=pl.ANY),
            pl.BlockSpec(memory_space=pltpu.MemorySpace.VMEM),
            pl.BlockSpec(memory_space=pltpu.MemorySpace.VMEM),
            pl.BlockSpec(memory_space=pltpu.MemorySpace.VMEM),
            pl.BlockSpec(memory_space=pltpu.MemorySpace.VMEM),
        ],
        out_specs=pl.BlockSpec(memory_space=pl.ANY),
        out_shape=jax.ShapeDtypeStruct((B, H), jnp.float32),
        scratch_shapes=[
            pltpu.VMEM((NB, BT, D), jnp.float32),
            pltpu.VMEM((NB, BT, H), jnp.float32),
            pltpu.SemaphoreType.DMA((NB,)),
            pltpu.SemaphoreType.DMA((NB,)),
        ],
    )


@jax.jit
def kernel(input_ids, table, gamma, beta, W, b):
    nb, seq = input_ids.shape
    V, D = table.shape
    H = W.shape[1]
    B = nb * seq
    rows = _make_sc_gather(nb, seq, V, D)(input_ids, table)
    out = _make_tc_proj(B, D, H, 256, 8)(rows, gamma, beta, W, b)
    return out.reshape(nb, seq, H)


# speedup vs baseline: 1.0733x; 1.0733x over previous
import functools

import jax
import jax.numpy as jnp
from jax import lax
from jax.experimental import pallas as pl
from jax.experimental.pallas import tpu as pltpu
from jax.experimental.pallas import tpu_sc as plsc

EPS = 1e-12


@functools.cache
def _make_sc_gather(nb, seq, V, D, Bc, off):
    info = plsc.get_sparse_core_info()
    NC, NS = info.num_cores, info.num_subcores
    NW = NC * NS
    assert Bc % NW == 0
    b_per_w = Bc // NW
    assert seq % b_per_w == 0
    CH = min(128, b_per_w)
    assert b_per_w % CH == 0
    n_ch = b_per_w // CH
    mesh = plsc.VectorSubcoreMesh(core_axis_name="c", subcore_axis_name="s")

    @functools.partial(
        pl.kernel,
        mesh=mesh,
        out_type=jax.ShapeDtypeStruct((Bc, D), jnp.float32),
        scratch_types=[
            pltpu.VMEM((b_per_w,), jnp.int32),
            pltpu.VMEM((b_per_w, D), jnp.float32),
        ]
        + [pltpu.SemaphoreType.DMA] * n_ch,
    )
    def gather(idx_hbm, table_hbm, out_hbm, idx_v, rows_v, *sems):
        wid = lax.axis_index("s") * NC + lax.axis_index("c")
        base = wid * b_per_w
        flat = off + base
        row = flat // seq
        col = flat % seq
        pltpu.sync_copy(idx_hbm.at[row, pl.ds(col, b_per_w)], idx_v)
        gathers = [
            pltpu.async_copy(
                table_hbm.at[idx_v.at[pl.ds(j * CH, CH)]],
                rows_v.at[pl.ds(j * CH, CH)],
                sems[j],
            )
            for j in range(n_ch)
        ]
        for g in gathers:
            g.wait()
        pltpu.sync_copy(rows_v, out_hbm.at[pl.ds(base, b_per_w)])

    return gather


def _proj_body(x_hbm, g_ref, bt_ref, w_ref, b_ref, o_hbm, xb, ob, sx, so,
               *, S, BT, NB, off):
    def xcopy(s):
        return pltpu.make_async_copy(
            x_hbm.at[pl.ds(s * BT, BT)], xb.at[s % NB], sx.at[s % NB]
        )

    def ocopy(s):
        return pltpu.make_async_copy(
            ob.at[s % NB], o_hbm.at[pl.ds(off + s * BT, BT)], so.at[s % NB]
        )

    for s in range(min(NB, S)):
        xcopy(s).start()
    gam = g_ref[...][None, :]
    bet = bt_ref[...][None, :]
    w = w_ref[...]
    bias = b_ref[...][None, :]
    for s in range(S):
        xcopy(s).wait()
        x = xb[s % NB]
        mean = jnp.mean(x, axis=1, keepdims=True)
        xc = x - mean
        var = jnp.mean(xc * xc, axis=1, keepdims=True)
        xn = xc * lax.rsqrt(var + EPS) * gam + bet
        if s >= NB:
            ocopy(s - NB).wait()
        ob[s % NB] = jnp.dot(xn, w, preferred_element_type=jnp.float32) + bias
        ocopy(s).start()
        if s + NB < S:
            xcopy(s + NB).start()
    for s in range(max(S - NB, 0), S):
        ocopy(s).wait()


@functools.cache
def _make_tc_proj(B, Bc, D, H, BT, NB, off, first):
    S = Bc // BT
    vmem = pltpu.MemorySpace.VMEM
    in_specs = [
        pl.BlockSpec(memory_space=pl.ANY),
        pl.BlockSpec(memory_space=vmem),
        pl.BlockSpec(memory_space=vmem),
        pl.BlockSpec(memory_space=vmem),
        pl.BlockSpec(memory_space=vmem),
    ]
    scratch = [
        pltpu.VMEM((NB, BT, D), jnp.float32),
        pltpu.VMEM((NB, BT, H), jnp.float32),
        pltpu.SemaphoreType.DMA((NB,)),
        pltpu.SemaphoreType.DMA((NB,)),
    ]
    out_shape = jax.ShapeDtypeStruct((B, H), jnp.float32)
    body = functools.partial(_proj_body, S=S, BT=BT, NB=NB, off=off)
    if first:
        return pl.pallas_call(
            body,
            in_specs=in_specs,
            out_specs=pl.BlockSpec(memory_space=pl.ANY),
            out_shape=out_shape,
            scratch_shapes=scratch,
        )

    def body_acc(x_hbm, g_ref, bt_ref, w_ref, b_ref, acc_ref, o_hbm, *sc):
        del acc_ref
        body(x_hbm, g_ref, bt_ref, w_ref, b_ref, o_hbm, *sc)

    return pl.pallas_call(
        body_acc,
        in_specs=in_specs + [pl.BlockSpec(memory_space=pl.ANY)],
        out_specs=pl.BlockSpec(memory_space=pl.ANY),
        out_shape=out_shape,
        scratch_shapes=scratch,
        input_output_aliases={5: 0},
    )


@jax.jit
def kernel(input_ids, table, gamma, beta, W, b):
    nb, seq = input_ids.shape
    V, D = table.shape
    H = W.shape[1]
    B = nb * seq
    C = 2
    BT = 512
    NB = 6
    Bc = B // C
    rows = [
        _make_sc_gather(nb, seq, V, D, Bc, c * Bc)(input_ids, table)
        for c in range(C)
    ]
    acc = _make_tc_proj(B, Bc, D, H, BT, NB, 0, True)(
        rows[0], gamma, beta, W, b
    )
    for c in range(1, C):
        acc = _make_tc_proj(B, Bc, D, H, BT, NB, c * Bc, False)(
            rows[c], gamma, beta, W, b, acc
        )
    return acc.reshape(nb, seq, H)


# speedup vs baseline: 1.1562x; 1.0772x over previous
import functools

import jax
import jax.numpy as jnp
from jax import lax
from jax.experimental import pallas as pl
from jax.experimental.pallas import tpu as pltpu
from jax.experimental.pallas import tpu_sc as plsc

EPS = 1e-12


@functools.cache
def _make_sc_gather(nb, seq, V, D, Bc, off):
    info = plsc.get_sparse_core_info()
    NC, NS = info.num_cores, info.num_subcores
    NW = NC * NS
    assert Bc % NW == 0
    b_per_w = Bc // NW
    assert seq % b_per_w == 0
    CH = min(128, b_per_w)
    assert b_per_w % CH == 0
    n_ch = b_per_w // CH
    mesh = plsc.VectorSubcoreMesh(core_axis_name="c", subcore_axis_name="s")

    @functools.partial(
        pl.kernel,
        mesh=mesh,
        out_type=jax.ShapeDtypeStruct((Bc, D), jnp.float32),
        scratch_types=[
            pltpu.VMEM((b_per_w,), jnp.int32),
            pltpu.VMEM((b_per_w, D), jnp.float32),
        ]
        + [pltpu.SemaphoreType.DMA] * n_ch,
    )
    def gather(idx_hbm, table_hbm, out_hbm, idx_v, rows_v, *sems):
        wid = lax.axis_index("s") * NC + lax.axis_index("c")
        base = wid * b_per_w
        flat = off + base
        row = flat // seq
        col = flat % seq
        pltpu.sync_copy(idx_hbm.at[row, pl.ds(col, b_per_w)], idx_v)
        gathers = [
            pltpu.async_copy(
                table_hbm.at[idx_v.at[pl.ds(j * CH, CH)]],
                rows_v.at[pl.ds(j * CH, CH)],
                sems[j],
            )
            for j in range(n_ch)
        ]
        for g in gathers:
            g.wait()
        pltpu.sync_copy(rows_v, out_hbm.at[pl.ds(base, b_per_w)])

    return gather


def _proj_body(x_hbm, g_ref, bt_ref, w_ref, b_ref, o_hbm, xb, ob, sx, so,
               *, S, BT, NB, off):
    def xcopy(s):
        return pltpu.make_async_copy(
            x_hbm.at[pl.ds(s * BT, BT)], xb.at[s % NB], sx.at[s % NB]
        )

    def ocopy(s):
        return pltpu.make_async_copy(
            ob.at[s % NB], o_hbm.at[pl.ds(off + s * BT, BT)], so.at[s % NB]
        )

    for s in range(min(NB, S)):
        xcopy(s).start()
    gam = g_ref[...][None, :]
    bet = bt_ref[...][None, :]
    w = w_ref[...]
    bias = b_ref[...][None, :]
    for s in range(S):
        xcopy(s).wait()
        x = xb[s % NB]
        mean = jnp.mean(x, axis=1, keepdims=True)
        xc = x - mean
        var = jnp.mean(xc * xc, axis=1, keepdims=True)
        xn = xc * lax.rsqrt(var + EPS) * gam + bet
        if s >= NB:
            ocopy(s - NB).wait()
        ob[s % NB] = jnp.dot(xn, w, preferred_element_type=jnp.float32) + bias
        ocopy(s).start()
        if s + NB < S:
            xcopy(s + NB).start()
    for s in range(max(S - NB, 0), S):
        ocopy(s).wait()


@functools.cache
def _make_tc_proj(B, Bc, D, H, BT, NB, off, first):
    S = Bc // BT
    vmem = pltpu.MemorySpace.VMEM
    in_specs = [
        pl.BlockSpec(memory_space=pl.ANY),
        pl.BlockSpec(memory_space=vmem),
        pl.BlockSpec(memory_space=vmem),
        pl.BlockSpec(memory_space=vmem),
        pl.BlockSpec(memory_space=vmem),
    ]
    scratch = [
        pltpu.VMEM((NB, BT, D), jnp.float32),
        pltpu.VMEM((NB, BT, H), jnp.float32),
        pltpu.SemaphoreType.DMA((NB,)),
        pltpu.SemaphoreType.DMA((NB,)),
    ]
    out_shape = jax.ShapeDtypeStruct((B, H), jnp.float32)
    body = functools.partial(_proj_body, S=S, BT=BT, NB=NB, off=off)
    if first:
        return pl.pallas_call(
            body,
            in_specs=in_specs,
            out_specs=pl.BlockSpec(memory_space=pl.ANY),
            out_shape=out_shape,
            scratch_shapes=scratch,
        )

    def body_acc(x_hbm, g_ref, bt_ref, w_ref, b_ref, acc_ref, o_hbm, *sc):
        del acc_ref
        body(x_hbm, g_ref, bt_ref, w_ref, b_ref, o_hbm, *sc)

    return pl.pallas_call(
        body_acc,
        in_specs=in_specs + [pl.BlockSpec(memory_space=pl.ANY)],
        out_specs=pl.BlockSpec(memory_space=pl.ANY),
        out_shape=out_shape,
        scratch_shapes=scratch,
        input_output_aliases={5: 0},
    )


@jax.jit
def kernel(input_ids, table, gamma, beta, W, b):
    nb, seq = input_ids.shape
    V, D = table.shape
    H = W.shape[1]
    B = nb * seq
    C = 1
    BT = 512
    NB = 6
    Bc = B // C
    rows = [
        _make_sc_gather(nb, seq, V, D, Bc, c * Bc)(input_ids, table)
        for c in range(C)
    ]
    acc = _make_tc_proj(B, Bc, D, H, BT, NB, 0, True)(
        rows[0], gamma, beta, W, b
    )
    for c in range(1, C):
        acc = _make_tc_proj(B, Bc, D, H, BT, NB, c * Bc, False)(
            rows[c], gamma, beta, W, b, acc
        )
    return acc.reshape(nb, seq, H)


# speedup vs baseline: 1.1624x; 1.0053x over previous
import functools

import jax
import jax.numpy as jnp
from jax import lax
from jax.experimental import pallas as pl
from jax.experimental.pallas import tpu as pltpu
from jax.experimental.pallas import tpu_sc as plsc

EPS = 1e-12


@functools.cache
def _make_sc_gather(nb, seq, V, D, Bc, off):
    info = plsc.get_sparse_core_info()
    NC, NS = info.num_cores, info.num_subcores
    NW = NC * NS
    assert Bc % NW == 0
    b_per_w = Bc // NW
    assert seq % b_per_w == 0
    CH = min(128, b_per_w)
    assert b_per_w % CH == 0
    n_ch = b_per_w // CH
    mesh = plsc.VectorSubcoreMesh(core_axis_name="c", subcore_axis_name="s")

    @functools.partial(
        pl.kernel,
        mesh=mesh,
        out_type=jax.ShapeDtypeStruct((Bc, D), jnp.float32),
        scratch_types=[
            pltpu.VMEM((b_per_w,), jnp.int32),
            pltpu.VMEM((b_per_w, D), jnp.float32),
        ]
        + [pltpu.SemaphoreType.DMA] * n_ch,
    )
    def gather(idx_hbm, table_hbm, out_hbm, idx_v, rows_v, *sems):
        wid = lax.axis_index("s") * NC + lax.axis_index("c")
        base = wid * b_per_w
        flat = off + base
        row = flat // seq
        col = flat % seq
        pltpu.sync_copy(idx_hbm.at[row, pl.ds(col, b_per_w)], idx_v)
        gathers = [
            pltpu.async_copy(
                table_hbm.at[idx_v.at[pl.ds(j * CH, CH)]],
                rows_v.at[pl.ds(j * CH, CH)],
                sems[j],
            )
            for j in range(n_ch)
        ]
        for g in gathers:
            g.wait()
        pltpu.sync_copy(rows_v, out_hbm.at[pl.ds(base, b_per_w)])

    return gather


def _proj_body(x_hbm, g_ref, bt_ref, w_ref, b_ref, o_hbm, xb, ob, sx, so,
               *, S, BT, NB, off):
    def xcopy(s):
        return pltpu.make_async_copy(
            x_hbm.at[pl.ds(s * BT, BT)], xb.at[s % NB], sx.at[s % NB]
        )

    def ocopy(s):
        return pltpu.make_async_copy(
            ob.at[s % NB], o_hbm.at[pl.ds(off + s * BT, BT)], so.at[s % NB]
        )

    for s in range(min(NB, S)):
        xcopy(s).start()
    gam = g_ref[...][None, :]
    bet = bt_ref[...][None, :]
    w = w_ref[...]
    bias = b_ref[...][None, :]
    for s in range(S):
        xcopy(s).wait()
        x = xb[s % NB]
        mean = jnp.mean(x, axis=1, keepdims=True)
        xc = x - mean
        var = jnp.mean(xc * xc, axis=1, keepdims=True)
        xn = xc * lax.rsqrt(var + EPS) * gam + bet
        if s >= NB:
            ocopy(s - NB).wait()
        ob[s % NB] = jnp.dot(xn, w, preferred_element_type=jnp.float32) + bias
        ocopy(s).start()
        if s + NB < S:
            xcopy(s + NB).start()
    for s in range(max(S - NB, 0), S):
        ocopy(s).wait()


@functools.cache
def _make_tc_proj(B, Bc, D, H, BT, NB, off, first):
    S = Bc // BT
    vmem = pltpu.MemorySpace.VMEM
    in_specs = [
        pl.BlockSpec(memory_space=pl.ANY),
        pl.BlockSpec(memory_space=vmem),
        pl.BlockSpec(memory_space=vmem),
        pl.BlockSpec(memory_space=vmem),
        pl.BlockSpec(memory_space=vmem),
    ]
    scratch = [
        pltpu.VMEM((NB, BT, D), jnp.float32),
        pltpu.VMEM((NB, BT, H), jnp.float32),
        pltpu.SemaphoreType.DMA((NB,)),
        pltpu.SemaphoreType.DMA((NB,)),
    ]
    out_shape = jax.ShapeDtypeStruct((B, H), jnp.float32)
    body = functools.partial(_proj_body, S=S, BT=BT, NB=NB, off=off)
    if first:
        return pl.pallas_call(
            body,
            in_specs=in_specs,
            out_specs=pl.BlockSpec(memory_space=pl.ANY),
            out_shape=out_shape,
            scratch_shapes=scratch,
        )

    def body_acc(x_hbm, g_ref, bt_ref, w_ref, b_ref, acc_ref, o_hbm, *sc):
        del acc_ref
        body(x_hbm, g_ref, bt_ref, w_ref, b_ref, o_hbm, *sc)

    return pl.pallas_call(
        body_acc,
        in_specs=in_specs + [pl.BlockSpec(memory_space=pl.ANY)],
        out_specs=pl.BlockSpec(memory_space=pl.ANY),
        out_shape=out_shape,
        scratch_shapes=scratch,
        input_output_aliases={5: 0},
    )


@jax.jit
def kernel(input_ids, table, gamma, beta, W, b):
    nb, seq = input_ids.shape
    V, D = table.shape
    H = W.shape[1]
    B = nb * seq
    C = 1
    BT = 512
    NB = 8
    Bc = B // C
    rows = [
        _make_sc_gather(nb, seq, V, D, Bc, c * Bc)(input_ids, table)
        for c in range(C)
    ]
    acc = _make_tc_proj(B, Bc, D, H, BT, NB, 0, True)(
        rows[0], gamma, beta, W, b
    )
    for c in range(1, C):
        acc = _make_tc_proj(B, Bc, D, H, BT, NB, c * Bc, False)(
            rows[c], gamma, beta, W, b, acc
        )
    return acc.reshape(nb, seq, H)
